# Initial kernel scaffold; baseline (speedup 1.0000x reference)
import jax
import jax.numpy as jnp
from jax.experimental import pallas as pl


def kernel(x, table):
    raise NotImplementedError("write your pallas kernel here")


# speedup vs baseline: 2.3886x; 2.3886x over previous
import functools

import numpy as np
import jax
import jax.numpy as jnp
from jax import lax
from jax.experimental import pallas as pl
from jax.experimental.pallas import tpu as pltpu
from jax.experimental.pallas import tpu_sc as plsc

_SEQ = 2048
_HID = 1024
_BATCH = 4
_ROPE_BASE = 10000.0

_NW = 32
_POS_W = _SEQ // _NW
_PCH = 16
_NCH = _POS_W // _PCH
_GRP = _HID // 16


def _rope_tables():
    i = np.arange(0, _HID, 2, dtype=np.float64)
    theta = _ROPE_BASE ** (-2.0 * i / _HID)
    pos = np.arange(_SEQ, dtype=np.float64)[:, None]
    m = pos * theta[None, :]
    cos = np.repeat(np.cos(m), 2, axis=1).astype(np.float32)
    sin = np.repeat(np.sin(m), 2, axis=1).astype(np.float32)
    sgn = np.where(np.arange(_HID) % 2 == 0, -1.0, 1.0).astype(np.float32)
    return cos, sin * sgn[None, :]


_COS_NP, _SSIN_NP = _rope_tables()

_mesh = plsc.VectorSubcoreMesh(core_axis_name="c", subcore_axis_name="s")


@functools.partial(
    pl.kernel,
    out_type=jax.ShapeDtypeStruct((_BATCH * _SEQ, _HID), jnp.float32),
    mesh=_mesh,
    scratch_types=[
        pltpu.VMEM((_PCH,), jnp.int32),
        pltpu.VMEM((_PCH, _HID), jnp.float32),
        pltpu.VMEM((_PCH, _HID), jnp.float32),
        pltpu.VMEM((_PCH, _HID), jnp.float32),
        pltpu.SemaphoreType.DMA,
    ],
)
def _rope_sc(x_hbm, tab_hbm, cos_hbm, sin_hbm, out_hbm,
             idx_v, rows_v, cos_v, sin_v, sem):
    wid = lax.axis_index("s") * 2 + lax.axis_index("c")
    perm = lax.iota(jnp.int32, 16) ^ 1
    for c in range(_NCH):
        p0 = wid * _POS_W + c * _PCH
        pltpu.sync_copy(cos_hbm.at[pl.ds(p0, _PCH)], cos_v)
        pltpu.sync_copy(sin_hbm.at[pl.ds(p0, _PCH)], sin_v)
        for b in range(_BATCH):
            base = b * _SEQ + p0
            pltpu.sync_copy(x_hbm.at[pl.ds(base, _PCH)], idx_v)
            pltpu.async_copy(tab_hbm.at[idx_v], rows_v, sem).wait()

            @plsc.parallel_loop(0, _PCH * _GRP, unroll=8)
            def _(i):
                r = i // _GRP
                g = i - r * _GRP
                sl = pl.ds(g * 16, 16)
                e = rows_v[r, sl]
                swp = e.at[perm].get(mode="promise_in_bounds")
                rows_v[r, sl] = e * cos_v[r, sl] + swp * sin_v[r, sl]

            pltpu.sync_copy(rows_v, out_hbm.at[pl.ds(base, _PCH)])


def kernel(x, table):
    cos = jnp.asarray(_COS_NP)
    ssin = jnp.asarray(_SSIN_NP)
    out = _rope_sc(x.reshape(-1), table, cos, ssin)
    return out.reshape(_BATCH, _SEQ, _HID)


# speedup vs baseline: 3.4282x; 1.4352x over previous
import functools

import numpy as np
import jax
import jax.numpy as jnp
from jax import lax
from jax.experimental import pallas as pl
from jax.experimental.pallas import tpu as pltpu
from jax.experimental.pallas import tpu_sc as plsc

_SEQ = 2048
_HID = 1024
_BATCH = 4
_ROPE_BASE = 10000.0

_NW = 32
_POS_W = _SEQ // _NW
_PCH = 16
_NCH = _POS_W // _PCH
_NBLK = _NCH * _BATCH
_GRP = _HID // 16


def _rope_table_packed():
    i = np.arange(0, _HID, 2, dtype=np.float64)
    theta = _ROPE_BASE ** (-2.0 * i / _HID)
    m = np.arange(_SEQ, dtype=np.float64)[:, None] * theta[None, :]
    packed = np.empty((_SEQ, _HID), dtype=np.float32)
    packed[:, 0::2] = np.cos(m)
    packed[:, 1::2] = np.sin(m)
    return packed


_PACKED_NP = _rope_table_packed()

_mesh = plsc.VectorSubcoreMesh(core_axis_name="c", subcore_axis_name="s")


@functools.partial(
    pl.kernel,
    out_type=jax.ShapeDtypeStruct((_BATCH * _SEQ, _HID), jnp.float32),
    mesh=_mesh,
    scratch_types=[
        pltpu.VMEM((_POS_W, _HID), jnp.float32),
        pltpu.VMEM((_PCH,), jnp.int32),
        pltpu.VMEM((_PCH,), jnp.int32),
        pltpu.VMEM((_PCH, _HID), jnp.float32),
        pltpu.VMEM((_PCH, _HID), jnp.float32),
        pltpu.SemaphoreType.DMA,
        pltpu.SemaphoreType.DMA,
        pltpu.SemaphoreType.DMA,
        pltpu.SemaphoreType.DMA,
        pltpu.SemaphoreType.DMA,
    ],
)
def _rope_sc(x_hbm, tab_hbm, p_hbm, out_hbm,
             p_v, idx0, idx1, rows0, rows1,
             psem, gsem0, gsem1, ssem0, ssem1):
    wid = lax.axis_index("s") * 2 + lax.axis_index("c")
    iota = lax.iota(jnp.int32, 16)
    perm = iota ^ 1
    cidx = iota & ~1
    sidx = iota | 1
    sgn = jnp.where((iota & 1) == 0, -1.0, 1.0).astype(jnp.float32)

    idx_b = (idx0, idx1)
    rows_b = (rows0, rows1)
    gsem_b = (gsem0, gsem1)
    ssem_b = (ssem0, ssem1)

    pos0 = wid * _POS_W
    pcopy = pltpu.async_copy(p_hbm.at[pl.ds(pos0, _POS_W)], p_v, psem)

    def blk_base(k):
        c, b = divmod(k, _BATCH)
        return b * _SEQ + pos0 + c * _PCH

    def start_gather(k):
        pltpu.sync_copy(x_hbm.at[pl.ds(blk_base(k), _PCH)], idx_b[k % 2])
        return pltpu.async_copy(tab_hbm.at[idx_b[k % 2]], rows_b[k % 2],
                                gsem_b[k % 2])

    gh = {0: start_gather(0)}
    sh = {}
    pcopy.wait()
    for k in range(_NBLK):
        if k + 1 < _NBLK:
            if k - 1 in sh:
                sh.pop(k - 1).wait()
            gh[k + 1] = start_gather(k + 1)
        gh.pop(k).wait()

        rows_v = rows_b[k % 2]
        c_blk = k // _BATCH

        @plsc.parallel_loop(0, _PCH * _GRP, unroll=8)
        def _(i):
            r = i // _GRP
            g = i - r * _GRP
            sl = pl.ds(g * 16, 16)
            e = rows_v[r, sl]
            t = p_v[c_blk * _PCH + r, sl]
            cv = t.at[cidx].get(mode="promise_in_bounds")
            sv = t.at[sidx].get(mode="promise_in_bounds")
            swp = e.at[perm].get(mode="promise_in_bounds")
            rows_v[r, sl] = e * cv + swp * (sgn * sv)

        sh[k] = pltpu.async_copy(rows_v, out_hbm.at[pl.ds(blk_base(k), _PCH)],
                                 ssem_b[k % 2])
    sh.pop(_NBLK - 2).wait()
    sh.pop(_NBLK - 1).wait()


def kernel(x, table):
    out = _rope_sc(x.reshape(-1), table, jnp.asarray(_PACKED_NP))
    return out.reshape(_BATCH, _SEQ, _HID)


# speedup vs baseline: 4.0400x; 1.1785x over previous
import functools

import numpy as np
import jax
import jax.numpy as jnp
from jax import lax
from jax.experimental import pallas as pl
from jax.experimental.pallas import tpu as pltpu
from jax.experimental.pallas import tpu_sc as plsc

_SEQ = 2048
_HID = 1024
_BATCH = 4
_ROPE_BASE = 10000.0

_NW = 32
_POS_W = _SEQ // _NW
_PCH = 16
_NCH = _POS_W // _PCH
_NBLK = _NCH * _BATCH
_GRP = _HID // 16
_NBUF = 3


def _rope_table_packed():
    i = np.arange(0, _HID, 2, dtype=np.float64)
    theta = _ROPE_BASE ** (-2.0 * i / _HID)
    m = np.arange(_SEQ, dtype=np.float64)[:, None] * theta[None, :]
    packed = np.empty((_SEQ, _HID), dtype=np.float32)
    packed[:, 0::2] = np.cos(m)
    packed[:, 1::2] = np.sin(m)
    return packed.reshape(-1)


_PACKED_NP = _rope_table_packed()

_mesh = plsc.VectorSubcoreMesh(core_axis_name="c", subcore_axis_name="s")


@functools.partial(
    pl.kernel,
    out_type=jax.ShapeDtypeStruct((_BATCH * _SEQ, _HID), jnp.float32),
    mesh=_mesh,
    scratch_types=[
        pltpu.VMEM((_POS_W * _HID,), jnp.float32),
        pltpu.VMEM((_BATCH, _POS_W), jnp.int32),
        pltpu.VMEM((_NBUF, _PCH, _HID), jnp.float32),
        pltpu.SemaphoreType.DMA,
        pltpu.SemaphoreType.DMA,
        pltpu.SemaphoreType.DMA,
        pltpu.SemaphoreType.DMA,
        pltpu.SemaphoreType.DMA,
        pltpu.SemaphoreType.DMA,
        pltpu.SemaphoreType.DMA,
        pltpu.SemaphoreType.DMA,
    ],
)
def _rope_sc(x_hbm, tab_hbm, p_hbm, out_hbm,
             p_v, idx_v, rows_v,
             psem, isem, gsem0, gsem1, gsem2, ssem0, ssem1, ssem2):
    wid = lax.axis_index("s") * 2 + lax.axis_index("c")
    iota = lax.iota(jnp.int32, 16)
    perm = iota ^ 1
    cidx = iota & ~1
    sidx = iota | 1
    sgn = jnp.where((iota & 1) == 0, -1.0, 1.0).astype(jnp.float32)

    gsem_b = (gsem0, gsem1, gsem2)
    ssem_b = (ssem0, ssem1, ssem2)

    pos0 = wid * _POS_W
    pcopy = pltpu.async_copy(p_hbm.at[pl.ds(pos0 * _HID, _POS_W * _HID)],
                             p_v, psem)
    icopies = [
        pltpu.async_copy(x_hbm.at[pl.ds(b * _SEQ + pos0, _POS_W)],
                         idx_v.at[b], isem)
        for b in range(_BATCH)
    ]

    def blk_base(k):
        c, b = divmod(k, _BATCH)
        return b * _SEQ + pos0 + c * _PCH

    def start_gather(k):
        c, b = divmod(k, _BATCH)
        return pltpu.async_copy(
            tab_hbm.at[idx_v.at[b, pl.ds(c * _PCH, _PCH)]],
            rows_v.at[k % _NBUF], gsem_b[k % _NBUF])

    for ic in icopies:
        ic.wait()
    gh = {0: start_gather(0), 1: start_gather(1)}
    sh = {}
    pcopy.wait()
    for k in range(_NBLK):
        if k + 2 < _NBLK:
            if k - 1 in sh:
                sh.pop(k - 1).wait()
            gh[k + 2] = start_gather(k + 2)
        gh.pop(k).wait()

        buf = rows_v.at[k % _NBUF]
        c_blk = k // _BATCH

        @plsc.parallel_loop(0, _PCH * _GRP, unroll=8)
        def _(i):
            r = i // _GRP
            g = i - r * _GRP
            sl = pl.ds(g * 16, 16)
            e = buf[r, sl]
            t = p_v[pl.ds(((c_blk * _PCH + r) * _GRP + g) * 16, 16)]
            cv = t.at[cidx].get(mode="promise_in_bounds")
            sv = t.at[sidx].get(mode="promise_in_bounds")
            swp = e.at[perm].get(mode="promise_in_bounds")
            buf[r, sl] = e * cv + swp * (sgn * sv)

        sh[k] = pltpu.async_copy(buf, out_hbm.at[pl.ds(blk_base(k), _PCH)],
                                 ssem_b[k % _NBUF])
    sh.pop(_NBLK - 2).wait()
    sh.pop(_NBLK - 1).wait()


def kernel(x, table):
    out = _rope_sc(x.reshape(-1), table, jnp.asarray(_PACKED_NP))
    return out.reshape(_BATCH, _SEQ, _HID)
